# baseline (speedup 1.0000x reference)
import jax
import jax.numpy as jnp
from jax import lax
from jax.experimental import pallas as pl
from jax.experimental.pallas import tpu as pltpu

_B, _FP, _NB = 4096, 4096, 1000
_MARGIN = 5
_NPAD = 1024
_BM = 256
_BK = 512


def _prep_body(wf_ref, wb_ref, wg_ref, bb_ref,
               wf16_ref, wbj_ref, wg16_ref, bbf_ref):
    wf16_ref[...] = wf_ref[...].astype(jnp.bfloat16)
    wg16_ref[...] = wg_ref[...].astype(jnp.bfloat16)
    r = lax.broadcasted_iota(jnp.int32, (_NB, _NPAD), 0)
    c = lax.broadcasted_iota(jnp.int32, (_NB, _NPAD), 1)
    j_mat = (r + c == _NB - 1).astype(jnp.bfloat16)
    wb16 = wb_ref[...].astype(jnp.bfloat16)
    wbj_ref[...] = jnp.dot(wb16, j_mat,
                           preferred_element_type=jnp.float32).astype(jnp.bfloat16)
    bbf_ref[...] = jnp.dot(bb_ref[...].astype(jnp.bfloat16), j_mat,
                           preferred_element_type=jnp.float32)


def _fused_body(a_ref, mw_ref, wf_ref, wb_ref, wg_ref, bf_ref, bb_ref, bg_ref,
                pred_ref, raw_ref):
    a = a_ref[...].astype(jnp.bfloat16)
    rev = jnp.dot(a, wb_ref[...], preferred_element_type=jnp.float32) + bb_ref[...]

    s = jnp.round(mw_ref[...]).astype(jnp.int32) + _MARGIN
    shift = jnp.clip((_NB - 1) - s, 0, _NPAD - 1)
    for k in range(_NPAD.bit_length() - 1):
        amt = 1 << k
        rolled = pltpu.roll(rev, _NPAD - amt, axis=1)
        rev = jnp.where((shift & amt) != 0, rolled, rev)

    fwd = jnp.dot(a, wf_ref[...], preferred_element_type=jnp.float32) + bf_ref[...]
    gate = jax.nn.sigmoid(
        jnp.dot(a, wg_ref[...], preferred_element_type=jnp.float32) + bg_ref[...])

    col = lax.broadcasted_iota(jnp.int32, (_BM, _NB), 1)
    bwd_rev = rev[:, :_NB]
    mask = col <= s
    raw = jnp.where(mask, gate * fwd + (1.0 - gate) * bwd_rev, 0.0)
    pred_ref[...] = jnp.maximum(raw, 0.0)
    raw_ref[...] = raw


@jax.jit
def _run(fingerprint, molecule_weight, W_fwd, b_fwd, W_bwd, b_bwd, W_gate, b_gate):
    full = lambda i: (0, 0)
    wf16, wbj, wg16, bbf = pl.pallas_call(
        _prep_body,
        grid=(_FP // _BK,),
        in_specs=[
            pl.BlockSpec((_BK, _NB), lambda i: (i, 0)),
            pl.BlockSpec((_BK, _NB), lambda i: (i, 0)),
            pl.BlockSpec((_BK, _NB), lambda i: (i, 0)),
            pl.BlockSpec((1, _NB), full),
        ],
        out_specs=[
            pl.BlockSpec((_BK, _NB), lambda i: (i, 0)),
            pl.BlockSpec((_BK, _NPAD), lambda i: (i, 0)),
            pl.BlockSpec((_BK, _NB), lambda i: (i, 0)),
            pl.BlockSpec((1, _NPAD), full),
        ],
        out_shape=[
            jax.ShapeDtypeStruct((_FP, _NB), jnp.bfloat16),
            jax.ShapeDtypeStruct((_FP, _NPAD), jnp.bfloat16),
            jax.ShapeDtypeStruct((_FP, _NB), jnp.bfloat16),
            jax.ShapeDtypeStruct((1, _NPAD), jnp.float32),
        ],
        compiler_params=pltpu.CompilerParams(
            dimension_semantics=("parallel",),
        ),
    )(W_fwd, W_bwd, W_gate, b_bwd.reshape(1, _NB))

    pred, raw = pl.pallas_call(
        _fused_body,
        grid=(_B // _BM,),
        in_specs=[
            pl.BlockSpec((_BM, _FP), lambda i: (i, 0)),
            pl.BlockSpec((_BM, 1), lambda i: (i, 0)),
            pl.BlockSpec((_FP, _NB), full),
            pl.BlockSpec((_FP, _NPAD), full),
            pl.BlockSpec((_FP, _NB), full),
            pl.BlockSpec((1, _NB), full),
            pl.BlockSpec((1, _NPAD), full),
            pl.BlockSpec((1, _NB), full),
        ],
        out_specs=[
            pl.BlockSpec((_BM, _NB), lambda i: (i, 0)),
            pl.BlockSpec((_BM, _NB), lambda i: (i, 0)),
        ],
        out_shape=[
            jax.ShapeDtypeStruct((_B, _NB), jnp.float32),
            jax.ShapeDtypeStruct((_B, _NB), jnp.float32),
        ],
        compiler_params=pltpu.CompilerParams(
            dimension_semantics=("parallel",),
        ),
    )(fingerprint, molecule_weight, wf16, wbj, wg16,
      b_fwd.reshape(1, _NB), bbf, b_gate.reshape(1, _NB))
    return (pred, raw)


def kernel(fingerprint, molecule_weight, W_fwd, b_fwd, W_bwd, b_bwd, W_gate, b_gate):
    return _run(fingerprint, molecule_weight, W_fwd, b_fwd, W_bwd, b_bwd,
                W_gate, b_gate)


# speedup vs baseline: 1.0569x; 1.0569x over previous
import jax
import jax.numpy as jnp
from jax import lax
from jax.experimental import pallas as pl
from jax.experimental.pallas import tpu as pltpu

_B, _FP, _NB = 4096, 4096, 1000
_MARGIN = 5
_NPAD = 1024
_BM = 256
_BK = 1024
_NCHUNK = _FP // _BK


def _body(a_ref, mw_ref, wf_hbm, wb_hbm, wg_hbm, bf_ref, bb_ref, bg_ref,
          pred_ref, raw_ref,
          wf16, wbj, wg16, bbf, stage0, stage1, sem0, sem1):

    @pl.when(pl.program_id(0) == 0)
    def _prep():
        r = lax.broadcasted_iota(jnp.int32, (_NB, _NPAD), 0)
        c = lax.broadcasted_iota(jnp.int32, (_NB, _NPAD), 1)
        j_mat = (r + c == _NB - 1).astype(jnp.bfloat16)
        bbf[...] = jnp.dot(bb_ref[...].astype(jnp.bfloat16), j_mat,
                           preferred_element_type=jnp.float32)

        stages = (stage0, stage1)
        sems = (sem0, sem1)
        tasks = []
        for w_hbm, dst, flip in ((wf_hbm, wf16, False), (wb_hbm, wbj, True),
                                 (wg_hbm, wg16, False)):
            for ci in range(_NCHUNK):
                tasks.append((w_hbm, dst, flip, ci))

        def _start(t, slot):
            w_hbm, _, _, ci = tasks[t]
            pltpu.make_async_copy(
                w_hbm.at[pl.ds(ci * _BK, _BK), :], stages[slot], sems[slot]
            ).start()

        _start(0, 0)
        for t in range(len(tasks)):
            slot = t % 2
            if t + 1 < len(tasks):
                _start(t + 1, (t + 1) % 2)
            w_hbm, dst, flip, ci = tasks[t]
            pltpu.make_async_copy(
                w_hbm.at[pl.ds(ci * _BK, _BK), :], stages[slot], sems[slot]
            ).wait()
            w16 = stages[slot][...].astype(jnp.bfloat16)
            if flip:
                dst[pl.ds(ci * _BK, _BK), :] = jnp.dot(
                    w16, j_mat, preferred_element_type=jnp.float32
                ).astype(jnp.bfloat16)
            else:
                dst[pl.ds(ci * _BK, _BK), :] = w16

    a = a_ref[...].astype(jnp.bfloat16)
    rev = jnp.dot(a, wbj[...], preferred_element_type=jnp.float32) + bbf[...]

    s = jnp.round(mw_ref[...]).astype(jnp.int32) + _MARGIN
    shift = jnp.clip((_NB - 1) - s, 0, _NPAD - 1)
    for k in range(_NPAD.bit_length() - 1):
        amt = 1 << k
        rolled = pltpu.roll(rev, _NPAD - amt, axis=1)
        rev = jnp.where((shift & amt) != 0, rolled, rev)

    fwd = jnp.dot(a, wf16[...], preferred_element_type=jnp.float32) + bf_ref[...]
    gate = jax.nn.sigmoid(
        jnp.dot(a, wg16[...], preferred_element_type=jnp.float32) + bg_ref[...])

    col = lax.broadcasted_iota(jnp.int32, (_BM, _NB), 1)
    mask = col <= s
    raw = jnp.where(mask, gate * fwd + (1.0 - gate) * rev[:, :_NB], 0.0)
    pred_ref[...] = jnp.maximum(raw, 0.0)
    raw_ref[...] = raw


@jax.jit
def _run(fingerprint, molecule_weight, W_fwd, b_fwd, W_bwd, b_bwd, W_gate, b_gate):
    full = lambda i: (0, 0)
    hbm = pl.BlockSpec(memory_space=pltpu.MemorySpace.HBM)
    pred, raw = pl.pallas_call(
        _body,
        grid=(_B // _BM,),
        in_specs=[
            pl.BlockSpec((_BM, _FP), lambda i: (i, 0)),
            pl.BlockSpec((_BM, 1), lambda i: (i, 0)),
            hbm,
            hbm,
            hbm,
            pl.BlockSpec((1, _NB), full),
            pl.BlockSpec((1, _NB), full),
            pl.BlockSpec((1, _NB), full),
        ],
        out_specs=[
            pl.BlockSpec((_BM, _NB), lambda i: (i, 0)),
            pl.BlockSpec((_BM, _NB), lambda i: (i, 0)),
        ],
        out_shape=[
            jax.ShapeDtypeStruct((_B, _NB), jnp.float32),
            jax.ShapeDtypeStruct((_B, _NB), jnp.float32),
        ],
        scratch_shapes=[
            pltpu.VMEM((_FP, _NB), jnp.bfloat16),
            pltpu.VMEM((_FP, _NPAD), jnp.bfloat16),
            pltpu.VMEM((_FP, _NB), jnp.bfloat16),
            pltpu.VMEM((1, _NPAD), jnp.float32),
            pltpu.VMEM((_BK, _NB), jnp.float32),
            pltpu.VMEM((_BK, _NB), jnp.float32),
            pltpu.SemaphoreType.DMA,
            pltpu.SemaphoreType.DMA,
        ],
        compiler_params=pltpu.CompilerParams(
            dimension_semantics=("arbitrary",),
        ),
    )(fingerprint, molecule_weight, W_fwd, W_bwd, W_gate,
      b_fwd.reshape(1, _NB), b_bwd.reshape(1, _NB), b_gate.reshape(1, _NB))
    return (pred, raw)


def kernel(fingerprint, molecule_weight, W_fwd, b_fwd, W_bwd, b_bwd, W_gate, b_gate):
    return _run(fingerprint, molecule_weight, W_fwd, b_fwd, W_bwd, b_bwd,
                W_gate, b_gate)


# speedup vs baseline: 1.0586x; 1.0016x over previous
import jax
import jax.numpy as jnp
from jax import lax
from jax.experimental import pallas as pl
from jax.experimental.pallas import tpu as pltpu

_B, _FP, _NB = 4096, 4096, 1000
_MARGIN = 5
_NPAD = 1024
_BM = 256
_BK = 1024
_NCHUNK = _FP // _BK


def _body(a_ref, mw_ref, wf_hbm, wb_hbm, wg_hbm, bf_ref, bb_ref, bg_ref,
          pred_ref, raw_ref,
          wf16, wbj, wg16, bbf, stage0, stage1, sem0, sem1):

    @pl.when(pl.program_id(0) == 0)
    def _prep():
        r = lax.broadcasted_iota(jnp.int32, (_NB, _NPAD), 0)
        c = lax.broadcasted_iota(jnp.int32, (_NB, _NPAD), 1)
        j_mat = (r + c == _NB - 1).astype(jnp.bfloat16)
        bbf[...] = jnp.dot(bb_ref[...].astype(jnp.bfloat16)[None, :], j_mat,
                           preferred_element_type=jnp.float32)

        stages = (stage0, stage1)
        sems = (sem0, sem1)
        tasks = []
        for w_hbm, dst, flip in ((wf_hbm, wf16, False), (wb_hbm, wbj, True),
                                 (wg_hbm, wg16, False)):
            for ci in range(_NCHUNK):
                tasks.append((w_hbm, dst, flip, ci))

        def _start(t, slot):
            w_hbm, _, _, ci = tasks[t]
            pltpu.make_async_copy(
                w_hbm.at[pl.ds(ci * _BK, _BK), :], stages[slot], sems[slot]
            ).start()

        _start(0, 0)
        for t in range(len(tasks)):
            slot = t % 2
            if t + 1 < len(tasks):
                _start(t + 1, (t + 1) % 2)
            w_hbm, dst, flip, ci = tasks[t]
            pltpu.make_async_copy(
                w_hbm.at[pl.ds(ci * _BK, _BK), :], stages[slot], sems[slot]
            ).wait()
            w16 = stages[slot][...].astype(jnp.bfloat16)
            if flip:
                dst[pl.ds(ci * _BK, _BK), :] = jnp.dot(
                    w16, j_mat, preferred_element_type=jnp.float32
                ).astype(jnp.bfloat16)
            else:
                dst[pl.ds(ci * _BK, _BK), :] = w16

    a = a_ref[...].astype(jnp.bfloat16)
    rev = jnp.dot(a, wbj[...], preferred_element_type=jnp.float32) + bbf[...]

    s = jnp.round(mw_ref[...]).astype(jnp.int32) + _MARGIN
    shift = jnp.clip((_NB - 1) - s, 0, _NPAD - 1)
    for k in range(_NPAD.bit_length() - 1):
        amt = 1 << k
        rolled = pltpu.roll(rev, _NPAD - amt, axis=1)
        rev = jnp.where((shift & amt) != 0, rolled, rev)

    fwd = jnp.dot(a, wf16[...], preferred_element_type=jnp.float32) + bf_ref[...][None, :]
    gate = jax.nn.sigmoid(
        jnp.dot(a, wg16[...], preferred_element_type=jnp.float32) + bg_ref[...][None, :])

    col = lax.broadcasted_iota(jnp.int32, (_BM, _NB), 1)
    mask = col <= s
    raw = jnp.where(mask, gate * fwd + (1.0 - gate) * rev[:, :_NB], 0.0)
    pred_ref[...] = jnp.maximum(raw, 0.0)
    raw_ref[...] = raw


@jax.jit
def _run(fingerprint, molecule_weight, W_fwd, b_fwd, W_bwd, b_bwd, W_gate, b_gate):
    full = lambda i: (0, 0)
    hbm = pl.BlockSpec(memory_space=pltpu.MemorySpace.HBM)
    pred, raw = pl.pallas_call(
        _body,
        grid=(_B // _BM,),
        in_specs=[
            pl.BlockSpec((_BM, _FP), lambda i: (i, 0)),
            pl.BlockSpec((_BM, 1), lambda i: (i, 0)),
            hbm,
            hbm,
            hbm,
            pl.BlockSpec((_NB,), lambda i: (0,)),
            pl.BlockSpec((_NB,), lambda i: (0,)),
            pl.BlockSpec((_NB,), lambda i: (0,)),
        ],
        out_specs=[
            pl.BlockSpec((_BM, _NB), lambda i: (i, 0)),
            pl.BlockSpec((_BM, _NB), lambda i: (i, 0)),
        ],
        out_shape=[
            jax.ShapeDtypeStruct((_B, _NB), jnp.float32),
            jax.ShapeDtypeStruct((_B, _NB), jnp.float32),
        ],
        scratch_shapes=[
            pltpu.VMEM((_FP, _NB), jnp.bfloat16),
            pltpu.VMEM((_FP, _NPAD), jnp.bfloat16),
            pltpu.VMEM((_FP, _NB), jnp.bfloat16),
            pltpu.VMEM((1, _NPAD), jnp.float32),
            pltpu.VMEM((_BK, _NB), jnp.float32),
            pltpu.VMEM((_BK, _NB), jnp.float32),
            pltpu.SemaphoreType.DMA,
            pltpu.SemaphoreType.DMA,
        ],
        compiler_params=pltpu.CompilerParams(
            dimension_semantics=("arbitrary",),
        ),
    )(fingerprint, molecule_weight, W_fwd, W_bwd, W_gate, b_fwd, b_bwd, b_gate)
    return (pred, raw)


def kernel(fingerprint, molecule_weight, W_fwd, b_fwd, W_bwd, b_bwd, W_gate, b_gate):
    return _run(fingerprint, molecule_weight, W_fwd, b_fwd, W_bwd, b_bwd,
                W_gate, b_gate)
